# baseline (speedup 1.0000x reference)
import functools

import jax
import jax.numpy as jnp
from jax import lax
from jax.experimental import pallas as pl
from jax.experimental.pallas import tpu as pltpu
from jax.experimental.pallas import tpu_sc as plsc

B_TOKENS = 1024 * 20
D = 1024

NC = 2
NS = 16
NW = NC * NS

CHUNK = 40
MM_BLOCK = 1280

SIZES = (1280, 5120, 5120, 5120, 2560, 1280)
assert sum(SIZES) == B_TOKENS


def _make_sc_gather(rows):
    b_per_w = rows // NW
    n_chunks = b_per_w // CHUNK

    @functools.partial(
        pl.kernel,
        mesh=plsc.VectorSubcoreMesh(core_axis_name="c", subcore_axis_name="s"),
        out_type=jax.ShapeDtypeStruct((rows, D), jnp.float32),
        scratch_types=[
            pltpu.VMEM((n_chunks, CHUNK), jnp.int32),
            pltpu.VMEM((CHUNK, D), jnp.float32),
            pltpu.SemaphoreType.DMA,
        ],
    )
    def sc_gather(ids_hbm, table_hbm, out_hbm, idx_v, rows_v, sem):
        wid = lax.axis_index("s") * NC + lax.axis_index("c")
        base = wid * b_per_w
        pltpu.sync_copy(ids_hbm.at[wid], idx_v)

        def body(j, carry):
            pltpu.async_copy(table_hbm.at[idx_v.at[j]], rows_v, sem).wait()
            pltpu.sync_copy(rows_v, out_hbm.at[pl.ds(base + j * CHUNK, CHUNK)])
            return carry

        lax.fori_loop(0, n_chunks, body, 0)

    return sc_gather


_SC_GATHERS = {rows: _make_sc_gather(rows) for rows in set(SIZES)}


def _mm_compute(x_ref, w_ref, b_ref, o_ref):
    acc = lax.dot_general(
        x_ref[...].astype(jnp.bfloat16), w_ref[...].astype(jnp.bfloat16),
        dimension_numbers=(((1,), (1,)), ((), ())),
        preferred_element_type=jnp.float32,
    )
    o_ref[...] = acc + b_ref[...]


def _mm_first(x, w, b, rows):
    return pl.pallas_call(
        _mm_compute,
        grid=(rows // MM_BLOCK,),
        in_specs=[
            pl.BlockSpec((MM_BLOCK, D), lambda i: (i, 0)),
            pl.BlockSpec((D, D), lambda i: (0, 0)),
            pl.BlockSpec((1, D), lambda i: (0, 0)),
        ],
        out_specs=pl.BlockSpec((MM_BLOCK, D), lambda i: (i, 0)),
        out_shape=jax.ShapeDtypeStruct((B_TOKENS, D), jnp.float32),
    )(x, w, b)


def _mm_chunk(prev, x, w, b, rows, row0):
    def body(prev_ref, x_ref, w_ref, b_ref, o_ref):
        del prev_ref
        _mm_compute(x_ref, w_ref, b_ref, o_ref)

    blk0 = row0 // MM_BLOCK
    return pl.pallas_call(
        body,
        grid=(rows // MM_BLOCK,),
        in_specs=[
            pl.BlockSpec(memory_space=pltpu.MemorySpace.HBM),
            pl.BlockSpec((MM_BLOCK, D), lambda i: (i, 0)),
            pl.BlockSpec((D, D), lambda i: (0, 0)),
            pl.BlockSpec((1, D), lambda i: (0, 0)),
        ],
        out_specs=pl.BlockSpec(
            (MM_BLOCK, D), lambda i, blk0=blk0: (blk0 + i, 0)
        ),
        out_shape=jax.ShapeDtypeStruct((B_TOKENS, D), jnp.float32),
        input_output_aliases={0: 0},
    )(prev, x, w, b)


def kernel(input_ids, embed_weight, proj_weight, proj_bias):
    b, l = input_ids.shape
    ids = input_ids.T.reshape(-1).astype(jnp.int32)
    bias2 = proj_bias.reshape(1, D)
    offs = [0]
    for r in SIZES:
        offs.append(offs[-1] + r)
    gathered = [
        _SC_GATHERS[rows](
            ids[offs[c]:offs[c + 1]].reshape(NW, rows // (NW * CHUNK), CHUNK),
            embed_weight,
        )
        for c, rows in enumerate(SIZES)
    ]
    out = _mm_first(gathered[0], proj_weight, bias2, SIZES[0])
    for c in range(1, len(SIZES)):
        out = _mm_chunk(out, gathered[c], proj_weight, bias2, SIZES[c], offs[c])
    return jnp.swapaxes(out.reshape(l, b, D), 0, 1)


# speedup vs baseline: 1.1038x; 1.1038x over previous
import functools

import jax
import jax.numpy as jnp
from jax import lax
from jax.experimental import pallas as pl
from jax.experimental.pallas import tpu as pltpu
from jax.experimental.pallas import tpu_sc as plsc

B_TOKENS = 1024 * 20
D = 1024

NC = 2
NS = 16
NW = NC * NS

CHUNK = 40
MM_BLOCK = 1280

SIZES = (2560, 7680, 7680, 2560)
assert sum(SIZES) == B_TOKENS


def _make_sc_gather(rows):
    b_per_w = rows // NW
    n_chunks = b_per_w // CHUNK

    @functools.partial(
        pl.kernel,
        mesh=plsc.VectorSubcoreMesh(core_axis_name="c", subcore_axis_name="s"),
        out_type=jax.ShapeDtypeStruct((rows, D), jnp.float32),
        scratch_types=[
            pltpu.VMEM((n_chunks, CHUNK), jnp.int32),
            pltpu.VMEM((2, CHUNK, D), jnp.float32),
            pltpu.SemaphoreType.DMA,
            pltpu.SemaphoreType.DMA,
        ],
    )
    def sc_gather(ids_hbm, table_hbm, out_hbm, idx_v, rows_v, sem0, sem1):
        wid = lax.axis_index("s") * NC + lax.axis_index("c")
        base = wid * b_per_w
        pltpu.sync_copy(ids_hbm.at[wid], idx_v)
        sems = (sem0, sem1)
        copies = [None, None]
        copies[0] = pltpu.async_copy(
            table_hbm.at[idx_v.at[0]], rows_v.at[0], sems[0]
        )
        for j in range(n_chunks):
            nxt = (j + 1) % 2
            if j + 1 < n_chunks:
                copies[nxt] = pltpu.async_copy(
                    table_hbm.at[idx_v.at[j + 1]], rows_v.at[nxt], sems[nxt]
                )
            copies[j % 2].wait()
            pltpu.sync_copy(
                rows_v.at[j % 2], out_hbm.at[pl.ds(base + j * CHUNK, CHUNK)]
            )

    return sc_gather


_SC_GATHERS = {rows: _make_sc_gather(rows) for rows in set(SIZES)}


def _mm_compute(x_ref, w_ref, b_ref, o_ref):
    acc = lax.dot_general(
        x_ref[...].astype(jnp.bfloat16), w_ref[...].astype(jnp.bfloat16),
        dimension_numbers=(((1,), (1,)), ((), ())),
        preferred_element_type=jnp.float32,
    )
    o_ref[...] = acc + b_ref[...]


def _mm_first(x, w, b, rows):
    return pl.pallas_call(
        _mm_compute,
        grid=(rows // MM_BLOCK,),
        in_specs=[
            pl.BlockSpec((MM_BLOCK, D), lambda i: (i, 0)),
            pl.BlockSpec((D, D), lambda i: (0, 0)),
            pl.BlockSpec((1, D), lambda i: (0, 0)),
        ],
        out_specs=pl.BlockSpec((MM_BLOCK, D), lambda i: (i, 0)),
        out_shape=jax.ShapeDtypeStruct((B_TOKENS, D), jnp.float32),
    )(x, w, b)


def _mm_chunk(prev, x, w, b, rows, row0):
    def body(prev_ref, x_ref, w_ref, b_ref, o_ref):
        del prev_ref
        _mm_compute(x_ref, w_ref, b_ref, o_ref)

    blk0 = row0 // MM_BLOCK
    return pl.pallas_call(
        body,
        grid=(rows // MM_BLOCK,),
        in_specs=[
            pl.BlockSpec(memory_space=pltpu.MemorySpace.HBM),
            pl.BlockSpec((MM_BLOCK, D), lambda i: (i, 0)),
            pl.BlockSpec((D, D), lambda i: (0, 0)),
            pl.BlockSpec((1, D), lambda i: (0, 0)),
        ],
        out_specs=pl.BlockSpec(
            (MM_BLOCK, D), lambda i, blk0=blk0: (blk0 + i, 0)
        ),
        out_shape=jax.ShapeDtypeStruct((B_TOKENS, D), jnp.float32),
        input_output_aliases={0: 0},
    )(prev, x, w, b)


def kernel(input_ids, embed_weight, proj_weight, proj_bias):
    b, l = input_ids.shape
    ids = input_ids.T.reshape(-1).astype(jnp.int32)
    bias2 = proj_bias.reshape(1, D)
    offs = [0]
    for r in SIZES:
        offs.append(offs[-1] + r)
    gathered = [
        _SC_GATHERS[rows](
            ids[offs[c]:offs[c + 1]].reshape(NW, rows // (NW * CHUNK), CHUNK),
            embed_weight,
        )
        for c, rows in enumerate(SIZES)
    ]
    out = _mm_first(gathered[0], proj_weight, bias2, SIZES[0])
    for c in range(1, len(SIZES)):
        out = _mm_chunk(out, gathered[c], proj_weight, bias2, SIZES[c], offs[c])
    return jnp.swapaxes(out.reshape(l, b, D), 0, 1)
